# baseline (speedup 1.0000x reference)
import functools
import math

import jax
import jax.numpy as jnp
import numpy as np
from jax import lax
from jax.experimental import pallas as pl
from jax.experimental.pallas import tpu as pltpu
from jax.experimental.pallas import tpu_sc as plsc

VOCAB = 100000
SEQ_LEN = 2048
EMB = 128
BATCH = 4

NC = 2
NS = 16
NW = NC * NS
ROWS = BATCH * SEQ_LEN
BPW = ROWS // NW
IDXW = 128
NIDX = BPW // IDXW
CH = 64
NCH = BPW // CH
LANES = 16
SPW = SEQ_LEN // (NW // BATCH)
SCALE = np.float32(math.sqrt(EMB))

_mesh = plsc.VectorSubcoreMesh(core_axis_name="c", subcore_axis_name="s")


@functools.partial(
    pl.kernel,
    out_type=jax.ShapeDtypeStruct((BATCH, SEQ_LEN, EMB), jnp.float32),
    mesh=_mesh,
    scratch_types=[
        pltpu.VMEM((NIDX, IDXW), jnp.int32),
        pltpu.VMEM((BPW, EMB), jnp.float32),
        pltpu.VMEM((BPW, EMB), jnp.float32),
        pltpu.SemaphoreType.DMA,
        pltpu.SemaphoreType.DMA,
        pltpu.SemaphoreType.DMA,
        pltpu.SemaphoreType.DMA,
        pltpu.SemaphoreType.DMA,
        pltpu.SemaphoreType.DMA,
        pltpu.SemaphoreType.DMA,
    ],
)
def _emb_kernel(idx_hbm, tok_hbm, pos_hbm, out_hbm, idx_v, rows_v, pos_v,
                sem_i, sem_p, g0, g1, g2, g3, sem_w):
    wid = lax.axis_index("s") * NC + lax.axis_index("c")
    b = wid // (NW // BATCH)
    s0 = lax.rem(wid * BPW, SEQ_LEN)

    c_idx = pltpu.async_copy(idx_hbm.at[pl.ds(wid * NIDX, NIDX)], idx_v, sem_i)
    c_pos = pltpu.async_copy(pos_hbm.at[pl.ds(s0, BPW)], pos_v, sem_p)
    c_idx.wait()

    gsems = (g0, g1, g2, g3)
    copies = []
    for j in range(NCH):
        irow, ioff = (j * CH) // IDXW, (j * CH) % IDXW
        copies.append(
            pltpu.async_copy(
                tok_hbm.at[idx_v.at[irow, pl.ds(ioff, CH)]],
                rows_v.at[pl.ds(j * CH, CH)],
                gsems[j],
            )
        )
    c_pos.wait()

    def compute_chunk(j):
        @plsc.parallel_loop(0, CH, 1, unroll=4)
        def body(i):
            row = j * CH + i
            for k in range(EMB // LANES):
                sl = pl.ds(k * LANES, LANES)
                rows_v[row, sl] = (rows_v[row, sl] + pos_v[row, sl]) * SCALE

    writes = []
    for j in range(NCH):
        copies[j].wait()
        compute_chunk(j)
        writes.append(
            pltpu.async_copy(
                rows_v.at[pl.ds(j * CH, CH)],
                out_hbm.at[b].at[pl.ds(s0 + j * CH, CH)],
                sem_w,
            )
        )
    for w in writes:
        w.wait()


def kernel(x_bs, tok_weight, pos_weight):
    idx = x_bs.reshape(NW * NIDX, IDXW)
    return _emb_kernel(idx, tok_weight, pos_weight)


# speedup vs baseline: 1.0019x; 1.0019x over previous
import functools
import math

import jax
import jax.numpy as jnp
import numpy as np
from jax import lax
from jax.experimental import pallas as pl
from jax.experimental.pallas import tpu as pltpu
from jax.experimental.pallas import tpu_sc as plsc

VOCAB = 100000
SEQ_LEN = 2048
EMB = 128
BATCH = 4

NC = 2
NS = 16
NW = NC * NS
ROWS = BATCH * SEQ_LEN
BPW = ROWS // NW
WPB = NW // BATCH
CH = 64
NCH = BPW // CH
LANES = 16
SCALE = np.float32(math.sqrt(EMB))

_mesh = plsc.VectorSubcoreMesh(core_axis_name="c", subcore_axis_name="s")


@functools.partial(
    pl.kernel,
    out_type=jax.ShapeDtypeStruct((BATCH, SEQ_LEN, EMB), jnp.float32),
    mesh=_mesh,
    scratch_types=[
        pltpu.VMEM((BPW,), jnp.int32),
        pltpu.VMEM((BPW, EMB), jnp.float32),
        pltpu.VMEM((BPW, EMB), jnp.float32),
        pltpu.SemaphoreType.DMA,
        pltpu.SemaphoreType.DMA,
        pltpu.SemaphoreType.DMA,
        pltpu.SemaphoreType.DMA,
        pltpu.SemaphoreType.DMA,
        pltpu.SemaphoreType.DMA,
        pltpu.SemaphoreType.DMA,
    ],
)
def _emb_kernel(idx_hbm, tok_hbm, pos_hbm, out_hbm, idx_v, rows_v, pos_v,
                sem_i, sem_p, g0, g1, g2, g3, sem_w):
    wid = lax.axis_index("s") * NC + lax.axis_index("c")
    b = wid // WPB
    s0 = lax.rem(wid, WPB) * BPW

    c_idx = pltpu.async_copy(idx_hbm.at[b].at[pl.ds(s0, BPW)], idx_v, sem_i)
    c_pos = pltpu.async_copy(pos_hbm.at[pl.ds(s0, BPW)], pos_v, sem_p)
    c_idx.wait()

    gsems = (g0, g1, g2, g3)
    copies = []
    for j in range(NCH):
        copies.append(
            pltpu.async_copy(
                tok_hbm.at[idx_v.at[pl.ds(j * CH, CH)]],
                rows_v.at[pl.ds(j * CH, CH)],
                gsems[j],
            )
        )
    c_pos.wait()

    def compute_chunk(j):
        @plsc.parallel_loop(0, CH, 1, unroll=4)
        def body(i):
            row = j * CH + i
            for k in range(EMB // LANES):
                sl = pl.ds(k * LANES, LANES)
                rows_v[row, sl] = (rows_v[row, sl] + pos_v[row, sl]) * SCALE

    writes = []
    for j in range(NCH):
        copies[j].wait()
        compute_chunk(j)
        writes.append(
            pltpu.async_copy(
                rows_v.at[pl.ds(j * CH, CH)],
                out_hbm.at[b].at[pl.ds(s0 + j * CH, CH)],
                sem_w,
            )
        )
    for w in writes:
        w.wait()


def kernel(x_bs, tok_weight, pos_weight):
    return _emb_kernel(x_bs, tok_weight, pos_weight)


# speedup vs baseline: 1.0644x; 1.0624x over previous
import functools
import math

import jax
import jax.numpy as jnp
import numpy as np
from jax import lax
from jax.experimental import pallas as pl
from jax.experimental.pallas import tpu as pltpu
from jax.experimental.pallas import tpu_sc as plsc

VOCAB = 100000
SEQ_LEN = 2048
EMB = 128
BATCH = 4

NC = 2
NS = 16
NW = NC * NS
ROWS = BATCH * SEQ_LEN
BPW = ROWS // NW
WPB = NW // BATCH
CH = 64
NCH = BPW // CH
LANES = 16
SCALE = np.float32(math.sqrt(EMB))

_mesh = plsc.VectorSubcoreMesh(core_axis_name="c", subcore_axis_name="s")


@functools.partial(
    pl.kernel,
    out_type=jax.ShapeDtypeStruct((BATCH, SEQ_LEN, EMB), jnp.float32),
    mesh=_mesh,
    scratch_types=[
        pltpu.VMEM((BPW,), jnp.int32),
        pltpu.VMEM((BPW, EMB), jnp.float32),
        pltpu.SemaphoreType.DMA,
        pltpu.SemaphoreType.DMA,
        pltpu.SemaphoreType.DMA,
        pltpu.SemaphoreType.DMA,
        pltpu.SemaphoreType.DMA,
        pltpu.SemaphoreType.DMA,
        pltpu.SemaphoreType.DMA,
        pltpu.SemaphoreType.DMA,
        pltpu.SemaphoreType.DMA,
        pltpu.SemaphoreType.DMA,
    ],
)
def _emb_kernel(idx_hbm, tok_hbm, pos_hbm, out_hbm, idx_v, rows_v,
                sem_i, p0, p1, p2, p3, g0, g1, g2, g3, sem_w):
    wid = lax.axis_index("s") * NC + lax.axis_index("c")
    b = wid // WPB
    s0 = lax.rem(wid, WPB) * BPW

    psems = (p0, p1, p2, p3)
    gsems = (g0, g1, g2, g3)

    c_idx = pltpu.async_copy(idx_hbm.at[b].at[pl.ds(s0, BPW)], idx_v, sem_i)
    pcopies = [
        pltpu.async_copy(
            pos_hbm.at[pl.ds(s0 + j * CH, CH)],
            rows_v.at[pl.ds(j * CH, CH)],
            psems[j],
        )
        for j in range(NCH)
    ]
    c_idx.wait()

    gcopies = []
    for j in range(NCH):
        pcopies[j].wait()
        gcopies.append(
            pltpu.async_copy(
                tok_hbm.at[idx_v.at[pl.ds(j * CH, CH)]],
                rows_v.at[pl.ds(j * CH, CH)],
                gsems[j],
                add=True,
            )
        )

    def compute_chunk(j):
        @plsc.parallel_loop(0, CH, 1, unroll=4)
        def body(i):
            row = j * CH + i
            for k in range(EMB // LANES):
                sl = pl.ds(k * LANES, LANES)
                rows_v[row, sl] = rows_v[row, sl] * SCALE

    writes = []
    for j in range(NCH):
        gcopies[j].wait()
        compute_chunk(j)
        writes.append(
            pltpu.async_copy(
                rows_v.at[pl.ds(j * CH, CH)],
                out_hbm.at[b].at[pl.ds(s0 + j * CH, CH)],
                sem_w,
            )
        )
    for w in writes:
        w.wait()


def kernel(x_bs, tok_weight, pos_weight):
    return _emb_kernel(x_bs, tok_weight, pos_weight)


# speedup vs baseline: 1.0701x; 1.0053x over previous
import functools
import math

import jax
import jax.numpy as jnp
import numpy as np
from jax import lax
from jax.experimental import pallas as pl
from jax.experimental.pallas import tpu as pltpu
from jax.experimental.pallas import tpu_sc as plsc

VOCAB = 100000
SEQ_LEN = 2048
EMB = 128
BATCH = 4

NC = 2
NS = 16
NW = NC * NS
ROWS = BATCH * SEQ_LEN
BPW = ROWS // NW
WPB = NW // BATCH
CH = 128
NCH = BPW // CH
LANES = 16
SCALE = np.float32(math.sqrt(EMB))

_mesh = plsc.VectorSubcoreMesh(core_axis_name="c", subcore_axis_name="s")


@functools.partial(
    pl.kernel,
    out_type=jax.ShapeDtypeStruct((BATCH, SEQ_LEN, EMB), jnp.float32),
    mesh=_mesh,
    scratch_types=[
        pltpu.VMEM((BPW,), jnp.int32),
        pltpu.VMEM((BPW, EMB), jnp.float32),
        pltpu.SemaphoreType.DMA,
        pltpu.SemaphoreType.DMA,
        pltpu.SemaphoreType.DMA,
        pltpu.SemaphoreType.DMA,
        pltpu.SemaphoreType.DMA,
        pltpu.SemaphoreType.DMA,
    ],
)
def _emb_kernel(idx_hbm, tok_hbm, pos_hbm, out_hbm, idx_v, rows_v,
                sem_i, p0, p1, g0, g1, sem_w):
    wid = lax.axis_index("s") * NC + lax.axis_index("c")
    b = wid // WPB
    s0 = lax.rem(wid, WPB) * BPW

    psems = (p0, p1)
    gsems = (g0, g1)

    c_idx = pltpu.async_copy(idx_hbm.at[b].at[pl.ds(s0, BPW)], idx_v, sem_i)
    pcopies = [
        pltpu.async_copy(
            pos_hbm.at[pl.ds(s0 + j * CH, CH)],
            rows_v.at[pl.ds(j * CH, CH)],
            psems[j],
        )
        for j in range(NCH)
    ]
    c_idx.wait()

    gcopies = []
    for j in range(NCH):
        pcopies[j].wait()
        gcopies.append(
            pltpu.async_copy(
                tok_hbm.at[idx_v.at[pl.ds(j * CH, CH)]],
                rows_v.at[pl.ds(j * CH, CH)],
                gsems[j],
                add=True,
            )
        )

    def compute_chunk(j):
        @plsc.parallel_loop(0, CH, 1, unroll=2)
        def body(i):
            row = j * CH + i
            for k in range(EMB // LANES):
                sl = pl.ds(k * LANES, LANES)
                rows_v[row, sl] = rows_v[row, sl] * SCALE

    writes = []
    for j in range(NCH):
        gcopies[j].wait()
        compute_chunk(j)
        writes.append(
            pltpu.async_copy(
                rows_v.at[pl.ds(j * CH, CH)],
                out_hbm.at[b].at[pl.ds(s0 + j * CH, CH)],
                sem_w,
            )
        )
    for w in writes:
        w.wait()


def kernel(x_bs, tok_weight, pos_weight):
    return _emb_kernel(x_bs, tok_weight, pos_weight)
